# Initial kernel scaffold; baseline (speedup 1.0000x reference)
import jax
import jax.numpy as jnp
from jax.experimental import pallas as pl


def kernel(x):
    raise NotImplementedError("write your pallas kernel here")


# speedup vs baseline: 35.0892x; 35.0892x over previous
import jax
import jax.numpy as jnp
from jax.experimental import pallas as pl
from jax.experimental.pallas import tpu as pltpu

_CH = 256
_HALF = 128
_ROWS = 1024


def _lrn_body(x_ref, band_ref, o_ref):
    x = x_ref[...]
    win = jax.lax.dot_general(
        x * x,
        band_ref[...],
        (((1,), (0,)), ((), ())),
        precision=jax.lax.Precision.HIGHEST,
        preferred_element_type=jnp.float32,
    )
    o_ref[...] = x * jax.lax.rsqrt(win)


def kernel(x):
    b, h, w, c = x.shape
    m = b * h * w
    xf = x.reshape(m, c)

    j = jax.lax.broadcasted_iota(jnp.int32, (c, c), 0)
    cc = jax.lax.broadcasted_iota(jnp.int32, (c, c), 1)
    band = ((j >= cc - _HALF) & (j <= cc + _HALF - 1)).astype(jnp.float32)

    out = pl.pallas_call(
        _lrn_body,
        grid=(m // _ROWS,),
        in_specs=[
            pl.BlockSpec((_ROWS, c), lambda i: (i, 0)),
            pl.BlockSpec((c, c), lambda i: (0, 0)),
        ],
        out_specs=pl.BlockSpec((_ROWS, c), lambda i: (i, 0)),
        out_shape=jax.ShapeDtypeStruct((m, c), x.dtype),
        compiler_params=pltpu.CompilerParams(
            dimension_semantics=("parallel",)
        ),
    )(xf, band)
    return out.reshape(b, h, w, c)


# speedup vs baseline: 44.7312x; 1.2748x over previous
import jax
import jax.numpy as jnp
from jax.experimental import pallas as pl
from jax.experimental.pallas import tpu as pltpu

_CH = 256
_HALF = 128
_ROWS = 2048


def _lrn_body(x_ref, band_ref, o_ref):
    x = x_ref[...]
    win = jax.lax.dot_general(
        x * x,
        band_ref[...],
        (((1,), (0,)), ((), ())),
        precision=jax.lax.Precision.HIGHEST,
        preferred_element_type=jnp.float32,
    )
    o_ref[...] = x * jax.lax.rsqrt(win)


def kernel(x):
    b, h, w, c = x.shape
    m = b * h * w
    xf = x.reshape(m, c)

    j = jax.lax.broadcasted_iota(jnp.int32, (c, c), 0)
    cc = jax.lax.broadcasted_iota(jnp.int32, (c, c), 1)
    band = ((j >= cc - _HALF) & (j <= cc + _HALF - 1)).astype(jnp.float32)

    out = pl.pallas_call(
        _lrn_body,
        grid=(m // _ROWS,),
        in_specs=[
            pl.BlockSpec((_ROWS, c), lambda i: (i, 0)),
            pl.BlockSpec((c, c), lambda i: (0, 0)),
        ],
        out_specs=pl.BlockSpec((_ROWS, c), lambda i: (i, 0)),
        out_shape=jax.ShapeDtypeStruct((m, c), x.dtype),
        compiler_params=pltpu.CompilerParams(
            dimension_semantics=("parallel",)
        ),
    )(xf, band)
    return out.reshape(b, h, w, c)


# speedup vs baseline: 63.4072x; 1.4175x over previous
import jax
import jax.numpy as jnp
from jax.experimental import pallas as pl
from jax.experimental.pallas import tpu as pltpu

_CH = 256
_HALF = 128
_ROWS = 2048


def _lrn_body(x_ref, band_ref, o_ref):
    x = x_ref[...]
    sq = (x * x).astype(jnp.bfloat16)
    win = jax.lax.dot_general(
        sq,
        band_ref[...],
        (((1,), (0,)), ((), ())),
        preferred_element_type=jnp.float32,
    )
    o_ref[...] = x * jax.lax.rsqrt(win)


def kernel(x):
    b, h, w, c = x.shape
    m = b * h * w
    xf = x.reshape(m, c)

    j = jax.lax.broadcasted_iota(jnp.int32, (c, c), 0)
    cc = jax.lax.broadcasted_iota(jnp.int32, (c, c), 1)
    band = ((j >= cc - _HALF) & (j <= cc + _HALF - 1)).astype(jnp.bfloat16)

    out = pl.pallas_call(
        _lrn_body,
        grid=(m // _ROWS,),
        in_specs=[
            pl.BlockSpec((_ROWS, c), lambda i: (i, 0)),
            pl.BlockSpec((c, c), lambda i: (0, 0)),
        ],
        out_specs=pl.BlockSpec((_ROWS, c), lambda i: (i, 0)),
        out_shape=jax.ShapeDtypeStruct((m, c), x.dtype),
        compiler_params=pltpu.CompilerParams(
            dimension_semantics=("parallel",)
        ),
    )(xf, band)
    return out.reshape(b, h, w, c)


# speedup vs baseline: 72.0010x; 1.1355x over previous
import jax
import jax.numpy as jnp
from jax.experimental import pallas as pl
from jax.experimental.pallas import tpu as pltpu

_CH = 256
_HALF = 128
_ROWS = 3584


def _lrn_body(x_ref, band_ref, o_ref):
    x = x_ref[...]
    sq = (x * x).astype(jnp.bfloat16)
    win = jax.lax.dot_general(
        sq,
        band_ref[...],
        (((1,), (0,)), ((), ())),
        preferred_element_type=jnp.float32,
    )
    o_ref[...] = x * jax.lax.rsqrt(win)


def kernel(x):
    b, h, w, c = x.shape
    m = b * h * w
    xf = x.reshape(m, c)

    j = jax.lax.broadcasted_iota(jnp.int32, (c, c), 0)
    cc = jax.lax.broadcasted_iota(jnp.int32, (c, c), 1)
    band = ((j >= cc - _HALF) & (j <= cc + _HALF - 1)).astype(jnp.bfloat16)

    out = pl.pallas_call(
        _lrn_body,
        grid=(m // _ROWS,),
        in_specs=[
            pl.BlockSpec((_ROWS, c), lambda i: (i, 0)),
            pl.BlockSpec((c, c), lambda i: (0, 0)),
        ],
        out_specs=pl.BlockSpec((_ROWS, c), lambda i: (i, 0)),
        out_shape=jax.ShapeDtypeStruct((m, c), x.dtype),
        compiler_params=pltpu.CompilerParams(
            dimension_semantics=("parallel",)
        ),
    )(xf, band)
    return out.reshape(b, h, w, c)


# speedup vs baseline: 74.7961x; 1.0388x over previous
import jax
import jax.numpy as jnp
from jax.experimental import pallas as pl
from jax.experimental.pallas import tpu as pltpu

_CH = 256
_HALF = 128
_ROWS = 7168


def _lrn_body(x_ref, band_ref, o_ref):
    x = x_ref[...]
    sq = (x * x).astype(jnp.bfloat16)
    win = jax.lax.dot_general(
        sq,
        band_ref[...],
        (((1,), (0,)), ((), ())),
        preferred_element_type=jnp.float32,
    )
    o_ref[...] = x * jax.lax.rsqrt(win)


def kernel(x):
    b, h, w, c = x.shape
    m = b * h * w
    xf = x.reshape(m, c)

    j = jax.lax.broadcasted_iota(jnp.int32, (c, c), 0)
    cc = jax.lax.broadcasted_iota(jnp.int32, (c, c), 1)
    band = ((j >= cc - _HALF) & (j <= cc + _HALF - 1)).astype(jnp.bfloat16)

    out = pl.pallas_call(
        _lrn_body,
        grid=(m // _ROWS,),
        in_specs=[
            pl.BlockSpec((_ROWS, c), lambda i: (i, 0)),
            pl.BlockSpec((c, c), lambda i: (0, 0)),
        ],
        out_specs=pl.BlockSpec((_ROWS, c), lambda i: (i, 0)),
        out_shape=jax.ShapeDtypeStruct((m, c), x.dtype),
        compiler_params=pltpu.CompilerParams(
            dimension_semantics=("parallel",)
        ),
    )(xf, band)
    return out.reshape(b, h, w, c)
